# baseline (speedup 1.0000x reference)
import functools

import jax
import jax.numpy as jnp
from jax import lax
from jax.experimental import pallas as pl
from jax.experimental.pallas import tpu as pltpu
from jax.experimental.pallas import tpu_sc as plsc

_EMBED = 128
_NAA = 22
_B = 4096 * 200


def _table_body(groups_ref, gemb_ref, aemb_ref, props_ref, w1_ref, b1_ref,
                w2_ref, b2_ref, wf_ref, bf_ref, gamma_ref, beta_ref,
                out_ref):
    hp = jax.lax.Precision.HIGHEST
    groups = jnp.broadcast_to(groups_ref[...], (8, _NAA))
    oh_t = (groups == lax.broadcasted_iota(jnp.int32, (8, _NAA), 0)
            ).astype(jnp.float32)
    gemb = jnp.concatenate(
        [gemb_ref[...], jnp.zeros((3, 64), jnp.float32)], axis=0)
    ge = lax.dot_general(oh_t, gemb, (((0,), (0,)), ((), ())),
                         precision=hp)
    h = jax.lax.dot(props_ref[...], w1_ref[...], precision=hp) + b1_ref[...]
    h = 0.5 * h * (1.0 + lax.erf(h * (2.0 ** -0.5)))
    pe = jax.lax.dot(h, w2_ref[...], precision=hp) + b2_ref[...]
    fused = (jax.lax.dot(ge, wf_ref[0:64], precision=hp)
             + jax.lax.dot(aemb_ref[...], wf_ref[64:128], precision=hp)
             + jax.lax.dot(pe, wf_ref[128:192], precision=hp)
             + bf_ref[...])
    mean = jnp.mean(fused, axis=-1, keepdims=True)
    var = jnp.mean((fused - mean) ** 2, axis=-1, keepdims=True)
    out_ref[...] = ((fused - mean) * lax.rsqrt(var + 1e-5)
                    * gamma_ref[...] + beta_ref[...])


def _build_table(groups, gemb, aemb, props, w1, b1, w2, b2, wf, bf,
                 gamma, beta):
    return pl.pallas_call(
        _table_body,
        out_shape=jax.ShapeDtypeStruct((_NAA, _EMBED), jnp.float32),
    )(groups, gemb, aemb, props, w1, b1, w2, b2, wf, bf, gamma, beta)


@functools.cache
def _make_gather():
    info = plsc.get_sparse_core_info()
    nw = info.num_cores * info.num_subcores
    b_per_w = _B // nw
    ch = 128
    k = 2
    n_ch = b_per_w // ch
    n_grp = n_ch // k
    mesh = plsc.VectorSubcoreMesh(core_axis_name="c", subcore_axis_name="s")

    @functools.partial(
        pl.kernel, mesh=mesh,
        out_type=jax.ShapeDtypeStruct((_B, _EMBED), jnp.float32),
        scratch_types=[
            pltpu.VMEM((b_per_w,), jnp.int32),
            pltpu.VMEM((3, k * ch, _EMBED), jnp.float32),
            pltpu.VMEM_SHARED((_NAA, _EMBED), jnp.float32),
            pltpu.SemaphoreType.DMA,
            pltpu.SemaphoreType.DMA,
            pltpu.SemaphoreType.DMA,
            pltpu.SemaphoreType.DMA,
        ],
    )
    def gather(table_hbm, idx_hbm, out_hbm, idx_v, rows_v, table_v,
               gsem, wsem0, wsem1, wsem2):
        sid = lax.axis_index("s")
        wid = sid * info.num_cores + lax.axis_index("c")
        base = wid * b_per_w
        wsems = (wsem0, wsem1, wsem2)
        @pl.when(sid == 0)
        def _():
            pltpu.sync_copy(table_hbm, table_v)
        pltpu.sync_copy(idx_hbm.at[pl.ds(base, b_per_w)], idx_v)
        plsc.subcore_barrier()

        def start_gathers(g, p):
            for b in range(k):
                pltpu.async_copy(
                    table_v.at[idx_v.at[pl.ds((g * k + b) * ch, ch)]],
                    rows_v.at[p, pl.ds(b * ch, ch)], gsem)

        def wait_gathers(g, p):
            for b in range(k):
                pltpu.make_async_copy(
                    table_v.at[idx_v.at[pl.ds((g * k + b) * ch, ch)]],
                    rows_v.at[p, pl.ds(b * ch, ch)], gsem).wait()

        def start_write(g, p):
            pltpu.async_copy(
                rows_v.at[p],
                out_hbm.at[pl.ds(base + g * k * ch, k * ch)], wsems[p])

        def wait_write(g, p):
            pltpu.make_async_copy(
                rows_v.at[p],
                out_hbm.at[pl.ds(base + g * k * ch, k * ch)],
                wsems[p]).wait()

        start_gathers(0, 0)
        start_gathers(1, 1)

        def body(gg, carry):
            for j in range(3):
                g = 3 * gg + j
                wait_gathers(g, j)
                start_write(g, j)

                @pl.when(g + 2 < n_grp)
                def _():
                    @pl.when(g >= 1)
                    def _():
                        wait_write(g - 1, (j - 1) % 3)
                    start_gathers(g + 2, (j + 2) % 3)
            return carry

        lax.fori_loop(0, n_grp // 3, body, 0, unroll=False)
        gl = n_grp - 1
        wait_gathers(gl, gl % 3)
        start_write(gl, gl % 3)
        for g in range(n_grp - 3, n_grp):
            wait_write(g, g % 3)

    return gather, nw, n_ch, ch


def kernel(indices, group_emb, aa_emb, W1, b1, W2, b2, Wf, bf, gamma, beta,
           aa_properties, aa_groups):
    table = _build_table(aa_groups.astype(jnp.int32), group_emb, aa_emb,
                         aa_properties, W1, b1, W2, b2, Wf, bf, gamma, beta)
    gather, nw, n_ch, ch = _make_gather()
    idx = indices.astype(jnp.int32).reshape(_B)
    out = gather(table, idx)
    return out.reshape(indices.shape[0], indices.shape[1], _EMBED)


# speedup vs baseline: 1.0045x; 1.0045x over previous
import functools

import jax
import jax.numpy as jnp
from jax import lax
from jax.experimental import pallas as pl
from jax.experimental.pallas import tpu as pltpu
from jax.experimental.pallas import tpu_sc as plsc

_EMBED = 128
_NAA = 22
_B = 4096 * 200


def _table_body(groups_ref, gemb_ref, aemb_ref, props_ref, w1_ref, b1_ref,
                w2_ref, b2_ref, wf_ref, bf_ref, gamma_ref, beta_ref,
                out_ref):
    hp = jax.lax.Precision.HIGHEST
    groups = jnp.broadcast_to(groups_ref[...], (8, _NAA))
    oh_t = (groups == lax.broadcasted_iota(jnp.int32, (8, _NAA), 0)
            ).astype(jnp.float32)
    gemb = jnp.concatenate(
        [gemb_ref[...], jnp.zeros((3, 64), jnp.float32)], axis=0)
    ge = lax.dot_general(oh_t, gemb, (((0,), (0,)), ((), ())),
                         precision=hp)
    h = jax.lax.dot(props_ref[...], w1_ref[...], precision=hp) + b1_ref[...]
    h = 0.5 * h * (1.0 + lax.erf(h * (2.0 ** -0.5)))
    pe = jax.lax.dot(h, w2_ref[...], precision=hp) + b2_ref[...]
    fused = (jax.lax.dot(ge, wf_ref[0:64], precision=hp)
             + jax.lax.dot(aemb_ref[...], wf_ref[64:128], precision=hp)
             + jax.lax.dot(pe, wf_ref[128:192], precision=hp)
             + bf_ref[...])
    mean = jnp.mean(fused, axis=-1, keepdims=True)
    var = jnp.mean((fused - mean) ** 2, axis=-1, keepdims=True)
    out_ref[...] = ((fused - mean) * lax.rsqrt(var + 1e-5)
                    * gamma_ref[...] + beta_ref[...])


def _build_table(groups, gemb, aemb, props, w1, b1, w2, b2, wf, bf,
                 gamma, beta):
    return pl.pallas_call(
        _table_body,
        out_shape=jax.ShapeDtypeStruct((_NAA, _EMBED), jnp.float32),
    )(groups, gemb, aemb, props, w1, b1, w2, b2, wf, bf, gamma, beta)


@functools.cache
def _make_gather():
    info = plsc.get_sparse_core_info()
    nw = info.num_cores * info.num_subcores
    b_per_w = _B // nw
    ch = 128
    k = 2
    n_ch = b_per_w // ch
    n_grp = n_ch // k
    mesh = plsc.VectorSubcoreMesh(core_axis_name="c", subcore_axis_name="s")

    @functools.partial(
        pl.kernel, mesh=mesh,
        out_type=jax.ShapeDtypeStruct((_B, _EMBED), jnp.float32),
        scratch_types=[
            pltpu.VMEM((n_ch, ch), jnp.int32),
            pltpu.VMEM((3, k * ch, _EMBED), jnp.float32),
            pltpu.VMEM_SHARED((_NAA, _EMBED), jnp.float32),
            pltpu.SemaphoreType.DMA,
            pltpu.SemaphoreType.DMA,
            pltpu.SemaphoreType.DMA,
            pltpu.SemaphoreType.DMA,
        ],
    )
    def gather(table_hbm, idx_hbm, out_hbm, idx_v, rows_v, table_v,
               gsem, wsem0, wsem1, wsem2):
        sid = lax.axis_index("s")
        wid = sid * info.num_cores + lax.axis_index("c")
        base = wid * b_per_w
        wsems = (wsem0, wsem1, wsem2)
        @pl.when(sid == 0)
        def _():
            pltpu.sync_copy(table_hbm, table_v)
        pltpu.sync_copy(idx_hbm.at[pl.ds(wid * n_ch, n_ch)], idx_v)
        plsc.subcore_barrier()

        def start_gathers(g, p):
            for b in range(k):
                pltpu.async_copy(
                    table_v.at[idx_v.at[g * k + b]],
                    rows_v.at[p, pl.ds(b * ch, ch)], gsem)

        def wait_gathers(g, p):
            for b in range(k):
                pltpu.make_async_copy(
                    table_v.at[idx_v.at[g * k + b]],
                    rows_v.at[p, pl.ds(b * ch, ch)], gsem).wait()

        def start_write(g, p):
            pltpu.async_copy(
                rows_v.at[p],
                out_hbm.at[pl.ds(base + g * k * ch, k * ch)], wsems[p])

        def wait_write(g, p):
            pltpu.make_async_copy(
                rows_v.at[p],
                out_hbm.at[pl.ds(base + g * k * ch, k * ch)],
                wsems[p]).wait()

        start_gathers(0, 0)
        start_gathers(1, 1)

        def body(gg, carry):
            for j in range(3):
                g = 3 * gg + j
                wait_gathers(g, j)
                start_write(g, j)

                @pl.when(g + 2 < n_grp)
                def _():
                    @pl.when(g >= 1)
                    def _():
                        wait_write(g - 1, (j - 1) % 3)
                    start_gathers(g + 2, (j + 2) % 3)
            return carry

        lax.fori_loop(0, n_grp // 3, body, 0, unroll=False)
        gl = n_grp - 1
        wait_gathers(gl, gl % 3)
        start_write(gl, gl % 3)
        for g in range(n_grp - 3, n_grp):
            wait_write(g, g % 3)

    return gather, nw, n_ch, ch


def kernel(indices, group_emb, aa_emb, W1, b1, W2, b2, Wf, bf, gamma, beta,
           aa_properties, aa_groups):
    table = _build_table(aa_groups.astype(jnp.int32), group_emb, aa_emb,
                         aa_properties, W1, b1, W2, b2, Wf, bf, gamma, beta)
    gather, nw, n_ch, ch = _make_gather()
    idx = indices.astype(jnp.int32).reshape(nw * n_ch, ch)
    out = gather(table, idx)
    return out.reshape(indices.shape[0], indices.shape[1], _EMBED)
